# baseline (speedup 1.0000x reference)
import jax
import jax.numpy as jnp
from jax import lax
from jax.experimental import pallas as pl

_TOKENS = 8192
_HIDDEN = 2048
_EXPERTS = 16
_TB = 512


def _body(x_ref, w_ref, b_ref, logits_ref, wpair_ref, ipair_ref, mask_ref):
    lg = lax.dot_general(
        x_ref[...], w_ref[...],
        (((1,), (1,)), ((), ())),
        preferred_element_type=jnp.float32,
    ) + b_ref[...]
    logits_ref[...] = lg

    lt = lg.T
    l0 = lt[0:1, :]
    l1 = lt[1:2, :]
    gt = l1 > l0
    m1 = jnp.where(gt, l1, l0)
    m2 = jnp.where(gt, l0, l1)
    i1 = jnp.where(gt, jnp.int32(1), jnp.int32(0))
    i2 = jnp.where(gt, jnp.int32(0), jnp.int32(1))
    for e in range(2, _EXPERTS):
        le = lt[e:e + 1, :]
        ev = jnp.int32(e)
        gt1 = le > m1
        gt2 = le > m2
        i2 = jnp.where(gt1, i1, jnp.where(gt2, ev, i2))
        m2 = jnp.where(gt1, m1, jnp.where(gt2, le, m2))
        i1 = jnp.where(gt1, ev, i1)
        m1 = jnp.where(gt1, le, m1)

    r = jnp.exp(m2 - m1)
    s = r + jnp.float32(1.0)
    w1 = jnp.float32(1.0) / s
    w2 = r / s

    wpair_ref[...] = jnp.concatenate([w1, w2], axis=0).T
    ipair_ref[...] = jnp.concatenate([i1, i2], axis=0).T

    eio = lax.broadcasted_iota(jnp.int32, (_EXPERTS, _TB), 0)
    mk1 = (eio == i1).astype(jnp.int32)
    mk2 = (eio == i2).astype(jnp.int32)
    mask_ref[...] = jnp.stack([mk1, mk2], axis=1)


_fused = pl.pallas_call(
    _body,
    grid=(_TOKENS // _TB,),
    in_specs=[
        pl.BlockSpec((_TB, _HIDDEN), lambda i: (i, 0)),
        pl.BlockSpec((_EXPERTS, _HIDDEN), lambda i: (0, 0)),
        pl.BlockSpec((1, _EXPERTS), lambda i: (0, 0)),
    ],
    out_specs=[
        pl.BlockSpec((_TB, _EXPERTS), lambda i: (i, 0)),
        pl.BlockSpec((_TB, 2), lambda i: (i, 0)),
        pl.BlockSpec((_TB, 2), lambda i: (i, 0)),
        pl.BlockSpec((_EXPERTS, 2, _TB), lambda i: (0, 0, i)),
    ],
    out_shape=[
        jax.ShapeDtypeStruct((_TOKENS, _EXPERTS), jnp.float32),
        jax.ShapeDtypeStruct((_TOKENS, 2), jnp.float32),
        jax.ShapeDtypeStruct((_TOKENS, 2), jnp.int32),
        jax.ShapeDtypeStruct((_EXPERTS, 2, _TOKENS), jnp.int32),
    ],
)


def kernel(x, W, b):
    logits, router_weight, select_idx, expert_mask = _fused(
        x, W, b.reshape(1, _EXPERTS))
    return (logits, router_weight, select_idx, expert_mask)


# speedup vs baseline: 1.0914x; 1.0914x over previous
import jax
import jax.numpy as jnp
from jax import lax
from jax.experimental import pallas as pl

_TOKENS = 8192
_HIDDEN = 2048
_EXPERTS = 16
_TB = 2048


def _body(x_ref, w_ref, b_ref, logits_ref, wpair_ref, ipair_ref, mask_ref):
    lg = lax.dot_general(
        x_ref[...], w_ref[...],
        (((1,), (1,)), ((), ())),
        preferred_element_type=jnp.float32,
    ) + b_ref[...]
    logits_ref[...] = lg

    lt = lg.T
    l0 = lt[0:1, :]
    l1 = lt[1:2, :]
    gt = l1 > l0
    m1 = jnp.where(gt, l1, l0)
    m2 = jnp.where(gt, l0, l1)
    i1 = jnp.where(gt, jnp.int32(1), jnp.int32(0))
    i2 = jnp.where(gt, jnp.int32(0), jnp.int32(1))
    for e in range(2, _EXPERTS):
        le = lt[e:e + 1, :]
        ev = jnp.int32(e)
        gt1 = le > m1
        gt2 = le > m2
        i2 = jnp.where(gt1, i1, jnp.where(gt2, ev, i2))
        m2 = jnp.where(gt1, m1, jnp.where(gt2, le, m2))
        i1 = jnp.where(gt1, ev, i1)
        m1 = jnp.where(gt1, le, m1)

    r = jnp.exp(m2 - m1)
    s = r + jnp.float32(1.0)
    w1 = jnp.float32(1.0) / s
    w2 = r / s

    wpair_ref[...] = jnp.concatenate([w1, w2], axis=0).T
    ipair_ref[...] = jnp.concatenate([i1, i2], axis=0).T

    eio = lax.broadcasted_iota(jnp.int32, (_EXPERTS, _TB), 0)
    mk1 = (eio == i1).astype(jnp.int32)
    mk2 = (eio == i2).astype(jnp.int32)
    mask_ref[...] = jnp.stack([mk1, mk2], axis=1)


_fused = pl.pallas_call(
    _body,
    grid=(_TOKENS // _TB,),
    in_specs=[
        pl.BlockSpec((_TB, _HIDDEN), lambda i: (i, 0)),
        pl.BlockSpec((_EXPERTS, _HIDDEN), lambda i: (0, 0)),
        pl.BlockSpec((1, _EXPERTS), lambda i: (0, 0)),
    ],
    out_specs=[
        pl.BlockSpec((_TB, _EXPERTS), lambda i: (i, 0)),
        pl.BlockSpec((_TB, 2), lambda i: (i, 0)),
        pl.BlockSpec((_TB, 2), lambda i: (i, 0)),
        pl.BlockSpec((_EXPERTS, 2, _TB), lambda i: (0, 0, i)),
    ],
    out_shape=[
        jax.ShapeDtypeStruct((_TOKENS, _EXPERTS), jnp.float32),
        jax.ShapeDtypeStruct((_TOKENS, 2), jnp.float32),
        jax.ShapeDtypeStruct((_TOKENS, 2), jnp.int32),
        jax.ShapeDtypeStruct((_EXPERTS, 2, _TOKENS), jnp.int32),
    ],
)


def kernel(x, W, b):
    logits, router_weight, select_idx, expert_mask = _fused(
        x, W, b.reshape(1, _EXPERTS))
    return (logits, router_weight, select_idx, expert_mask)
